# baseline (speedup 1.0000x reference)
import functools

import jax
import jax.numpy as jnp
from jax.experimental import pallas as pl
from jax.experimental.pallas import tpu as pltpu


def _prep_body(Cf, Ch, wc2_ref, wm_ref, wup_ref, bup_ref, bc_ref,
               p_ref, q_ref, wm2_o, ww_o, wcm_o, bt_o):
    wc2 = wc2_ref[...]
    wcm_o[...] = wc2[:, :Cf].astype(jnp.bfloat16)
    wc_up = wc2[:, Cf:Cf + Ch]
    bt_o[...] = jnp.dot(wc_up, bup_ref[...],
                        preferred_element_type=jnp.float32) + bc_ref[...]
    wm2_o[...] = jnp.dot(wm_ref[...].astype(jnp.bfloat16), p_ref[...],
                         preferred_element_type=jnp.float32).astype(jnp.bfloat16)
    wup_r = jnp.dot(wup_ref[...].astype(jnp.bfloat16), q_ref[...],
                    preferred_element_type=jnp.float32).astype(jnp.bfloat16)
    wcub = wc_up.astype(jnp.bfloat16)
    for p in range(4):
        sl = wup_r[:, p * Ch:(p + 1) * Ch]
        wp = jax.lax.dot_general(wcub, sl, (((1,), (1,)), ((), ())),
                                 preferred_element_type=jnp.float32)
        ww_o[p * Cf:(p + 1) * Cf, :] = wp.astype(jnp.bfloat16)


def _fused_body(R, W2, Cf, nt, x1_ref, x2m_ref, x2p_ref, x2n_ref,
                ww_ref, wm2_ref, wc_ref, s_ref, bm_ref, bt_ref,
                out_ref, win_ref, winl_ref, winr_ref):
    t = pl.program_id(1)
    TP = R * W2
    SZ = TP + 4 * W2
    zrow = jnp.zeros((Cf, W2), jnp.bfloat16)

    win_ref[:, 0:W2] = zrow
    win_ref[:, W2:2 * W2] = x2p_ref[0].astype(jnp.bfloat16)
    win_ref[:, 2 * W2:2 * W2 + TP] = x2m_ref[0].astype(jnp.bfloat16)
    win_ref[:, 2 * W2 + TP:3 * W2 + TP] = x2n_ref[0].astype(jnp.bfloat16)
    win_ref[:, 3 * W2 + TP:SZ] = zrow

    @pl.when(t == 0)
    def _():
        win_ref[:, W2:2 * W2] = zrow

    @pl.when(t == nt - 1)
    def _():
        win_ref[:, 2 * W2 + TP:3 * W2 + TP] = zrow

    ii = jax.lax.broadcasted_iota(jnp.int32, (1, SZ - 1), 1)
    ml = (((ii + 1) % W2) != 0).astype(jnp.bfloat16)
    mr = ((ii % W2) != (W2 - 1)).astype(jnp.bfloat16)
    winl_ref[:, 1:SZ] = win_ref[:, 0:SZ - 1] * ml
    winr_ref[:, 0:SZ - 1] = win_ref[:, 1:SZ] * mr

    acc = jnp.zeros((Cf, TP), jnp.float32)
    for ky in range(3):
        base = W2 + ky * W2
        for kx, src in ((0, winl_ref), (1, win_ref), (2, winr_ref)):
            tap = ky * 3 + kx
            acc = acc + jnp.dot(wm2_ref[:, tap * Cf:(tap + 1) * Cf],
                                src[:, base:base + TP],
                                preferred_element_type=jnp.float32)
    med = acc + bm_ref[...]
    med = jnp.maximum(med, 0.2 * med).astype(jnp.bfloat16)
    res = jnp.dot(wc_ref[...], med,
                  preferred_element_type=jnp.float32) + bt_ref[...]

    up4 = jnp.dot(ww_ref[...], x1_ref[0].astype(jnp.bfloat16),
                  preferred_element_type=jnp.float32)
    W = W2 // 2
    rows = []
    for r in range(R):
        dy, h = r % 2, r // 2
        a = up4[(2 * dy) * Cf:(2 * dy + 1) * Cf, h * W:(h + 1) * W]
        b = up4[(2 * dy + 1) * Cf:(2 * dy + 2) * Cf, h * W:(h + 1) * W]
        cat = jnp.concatenate([a, b], axis=1).astype(jnp.bfloat16)
        rows.append(jnp.dot(cat, s_ref[...],
                            preferred_element_type=jnp.float32))
    out_ref[0] = res + jnp.concatenate(rows, axis=1)


def kernel(x1, x2, w_up, b_up, w_med, b_med, w_c, b_c):
    B, Cin, H, W = x1.shape
    _, Cf, H2, W2 = x2.shape
    Ch = w_up.shape[1]
    R = next(r for r in (16, 8, 4, 2) if H2 % r == 0)
    nt = H2 // R
    TP = R * W2

    i9 = jnp.arange(9 * Cf)[:, None]
    j9 = jnp.arange(9 * Cf)[None, :]
    p_mat = ((i9 % 9 == j9 // Cf) & (i9 // 9 == j9 % Cf)).astype(jnp.bfloat16)
    i4 = jnp.arange(4 * Ch)[:, None]
    j4 = jnp.arange(4 * Ch)[None, :]
    q_mat = ((i4 % 4 == j4 // Ch) & (i4 // 4 == j4 % Ch)).astype(jnp.bfloat16)
    rr = jnp.arange(W2)[:, None]
    cc = jnp.arange(W2)[None, :]
    s_mat = (((cc % 2 == 0) & (rr == cc // 2))
             | ((cc % 2 == 1) & (rr == W2 // 2 + cc // 2))).astype(jnp.bfloat16)

    wm2, ww, wcm, bt = pl.pallas_call(
        functools.partial(_prep_body, Cf, Ch),
        out_shape=(
            jax.ShapeDtypeStruct((Cf, 9 * Cf), jnp.bfloat16),
            jax.ShapeDtypeStruct((4 * Cf, Cin), jnp.bfloat16),
            jax.ShapeDtypeStruct((Cf, Cf), jnp.bfloat16),
            jax.ShapeDtypeStruct((Cf, 1), jnp.float32),
        ),
        compiler_params=pltpu.CompilerParams(
            vmem_limit_bytes=64 * 1024 * 1024),
    )(w_c[:, :, 0, 0], w_med.reshape(Cf, Cf * 9), w_up.reshape(Cin, Ch * 4),
      b_up.reshape(Ch, 1), b_c.reshape(Cf, 1), p_mat, q_mat)

    x1f = x1.reshape(B, Cin, H * W)
    x2f = x2.reshape(B, Cf, H2 * W2)

    out = pl.pallas_call(
        functools.partial(_fused_body, R, W2, Cf, nt),
        out_shape=jax.ShapeDtypeStruct((B, Cf, H2 * W2), jnp.float32),
        grid=(B, nt),
        in_specs=[
            pl.BlockSpec((1, Cin, (R // 2) * W), lambda b, t: (b, 0, t)),
            pl.BlockSpec((1, Cf, TP), lambda b, t: (b, 0, t)),
            pl.BlockSpec((1, Cf, W2),
                         lambda b, t: (b, 0, jnp.maximum(R * t - 1, 0))),
            pl.BlockSpec((1, Cf, W2),
                         lambda b, t: (b, 0, jnp.minimum(R * (t + 1), H2 - 1))),
            pl.BlockSpec((4 * Cf, Cin), lambda b, t: (0, 0)),
            pl.BlockSpec((Cf, 9 * Cf), lambda b, t: (0, 0)),
            pl.BlockSpec((Cf, Cf), lambda b, t: (0, 0)),
            pl.BlockSpec((W2, W2), lambda b, t: (0, 0)),
            pl.BlockSpec((Cf, 1), lambda b, t: (0, 0)),
            pl.BlockSpec((Cf, 1), lambda b, t: (0, 0)),
        ],
        out_specs=pl.BlockSpec((1, Cf, TP), lambda b, t: (b, 0, t)),
        scratch_shapes=[pltpu.VMEM((Cf, TP + 4 * W2), jnp.bfloat16),
                        pltpu.VMEM((Cf, TP + 4 * W2), jnp.bfloat16),
                        pltpu.VMEM((Cf, TP + 4 * W2), jnp.bfloat16)],
        compiler_params=pltpu.CompilerParams(
            dimension_semantics=("parallel", "parallel"),
            vmem_limit_bytes=64 * 1024 * 1024),
    )(x1f, x2f, x2f, x2f, ww, wm2, wcm, s_mat,
      b_med.reshape(Cf, 1), bt)
    return out.reshape(B, Cf, H2, W2)


# speedup vs baseline: 1.5143x; 1.5143x over previous
import functools

import jax
import jax.numpy as jnp
from jax.experimental import pallas as pl
from jax.experimental.pallas import tpu as pltpu


def _fused_body(R, W2, Cf, nt, x1_ref, x2m_ref, x2p_ref, x2n_ref,
                ww_ref, wm_ref, wc_ref, s_ref, bm_ref, bt_ref,
                out_ref, win_ref, winl_ref, winr_ref):
    t = pl.program_id(1)
    TP = R * W2
    SZ = TP + 4 * W2
    zrow = jnp.zeros((Cf, W2), jnp.bfloat16)

    win_ref[:, 0:W2] = zrow
    win_ref[:, W2:2 * W2] = x2p_ref[0, :, 7, :].astype(jnp.bfloat16)
    win_ref[:, 2 * W2:2 * W2 + TP] = x2m_ref[0].reshape(Cf, TP).astype(jnp.bfloat16)
    win_ref[:, 2 * W2 + TP:3 * W2 + TP] = x2n_ref[0, :, 0, :].astype(jnp.bfloat16)
    win_ref[:, 3 * W2 + TP:SZ] = zrow

    @pl.when(t == 0)
    def _():
        win_ref[:, W2:2 * W2] = zrow

    @pl.when(t == nt - 1)
    def _():
        win_ref[:, 2 * W2 + TP:3 * W2 + TP] = zrow

    ii = jax.lax.broadcasted_iota(jnp.int32, (1, SZ - 1), 1)
    ml = (((ii + 1) % W2) != 0).astype(jnp.bfloat16)
    mr = ((ii % W2) != (W2 - 1)).astype(jnp.bfloat16)
    winl_ref[:, 1:SZ] = win_ref[:, 0:SZ - 1] * ml
    winr_ref[:, 0:SZ - 1] = win_ref[:, 1:SZ] * mr

    acc = jnp.zeros((Cf, TP), jnp.float32)
    for ky in range(3):
        base = W2 + ky * W2
        for kx, src in ((0, winl_ref), (1, win_ref), (2, winr_ref)):
            acc = acc + jnp.dot(wm_ref[ky * 3 + kx],
                                src[:, base:base + TP],
                                preferred_element_type=jnp.float32)
    med = acc + bm_ref[...]
    med = jnp.maximum(med, 0.2 * med).astype(jnp.bfloat16)
    res = jnp.dot(wc_ref[...], med,
                  preferred_element_type=jnp.float32) + bt_ref[...]

    up4 = jnp.dot(ww_ref[...], x1_ref[0].astype(jnp.bfloat16),
                  preferred_element_type=jnp.float32)
    W = W2 // 2
    for r in range(R):
        dy, h = r % 2, r // 2
        a = up4[(2 * dy) * Cf:(2 * dy + 1) * Cf, h * W:(h + 1) * W]
        b = up4[(2 * dy + 1) * Cf:(2 * dy + 2) * Cf, h * W:(h + 1) * W]
        cat = jnp.concatenate([a, b], axis=1).astype(jnp.bfloat16)
        row = jnp.dot(cat, s_ref[...], preferred_element_type=jnp.float32)
        out_ref[0, :, r, :] = res[:, r * W2:(r + 1) * W2] + row


def kernel(x1, x2, w_up, b_up, w_med, b_med, w_c, b_c):
    B, Cin, H, W = x1.shape
    _, Cf, H2, W2 = x2.shape
    R = 16 if H2 % 16 == 0 and H2 > 16 else 8
    nt = H2 // R
    TP = R * W2

    wc2 = w_c[:, :, 0, 0]
    wc_med = wc2[:, :Cf].astype(jnp.bfloat16)
    wc_up = wc2[:, Cf:]
    ww = jnp.einsum('oc,kcyx->yxok', wc_up, w_up).reshape(4 * Cf, Cin)
    ww = ww.astype(jnp.bfloat16)
    wm = jnp.transpose(w_med, (2, 3, 0, 1)).reshape(9, Cf, Cf)
    wm = wm.astype(jnp.bfloat16)
    bt = (b_c + wc_up @ b_up).reshape(Cf, 1)
    bm = b_med.reshape(Cf, 1)
    rr = jnp.arange(W2)[:, None]
    cc = jnp.arange(W2)[None, :]
    s_mat = (((cc % 2 == 0) & (rr == cc // 2))
             | ((cc % 2 == 1) & (rr == W2 // 2 + cc // 2))).astype(jnp.bfloat16)

    x1f = x1.reshape(B, Cin, H * W)

    out = pl.pallas_call(
        functools.partial(_fused_body, R, W2, Cf, nt),
        out_shape=jax.ShapeDtypeStruct((B, Cf, H2, W2), jnp.float32),
        grid=(B, nt),
        in_specs=[
            pl.BlockSpec((1, Cin, (R // 2) * W), lambda b, t: (b, 0, t)),
            pl.BlockSpec((1, Cf, R, W2), lambda b, t: (b, 0, t, 0)),
            pl.BlockSpec((1, Cf, 8, W2),
                         lambda b, t: (b, 0, jnp.maximum(R * t // 8 - 1, 0), 0)),
            pl.BlockSpec((1, Cf, 8, W2),
                         lambda b, t: (b, 0,
                                       jnp.minimum(R * (t + 1) // 8,
                                                   H2 // 8 - 1), 0)),
            pl.BlockSpec((4 * Cf, Cin), lambda b, t: (0, 0)),
            pl.BlockSpec((9, Cf, Cf), lambda b, t: (0, 0, 0)),
            pl.BlockSpec((Cf, Cf), lambda b, t: (0, 0)),
            pl.BlockSpec((W2, W2), lambda b, t: (0, 0)),
            pl.BlockSpec((Cf, 1), lambda b, t: (0, 0)),
            pl.BlockSpec((Cf, 1), lambda b, t: (0, 0)),
        ],
        out_specs=pl.BlockSpec((1, Cf, R, W2), lambda b, t: (b, 0, t, 0)),
        scratch_shapes=[pltpu.VMEM((Cf, TP + 4 * W2), jnp.bfloat16),
                        pltpu.VMEM((Cf, TP + 4 * W2), jnp.bfloat16),
                        pltpu.VMEM((Cf, TP + 4 * W2), jnp.bfloat16)],
        compiler_params=pltpu.CompilerParams(
            dimension_semantics=("parallel", "parallel"),
            vmem_limit_bytes=64 * 1024 * 1024),
    )(x1f, x2, x2, x2, ww, wm, wc_med, s_mat,
      b_med.reshape(Cf, 1), bt)
    return out


# speedup vs baseline: 1.6611x; 1.0969x over previous
import functools

import jax
import jax.numpy as jnp
from jax.experimental import pallas as pl
from jax.experimental.pallas import tpu as pltpu


def _fused_body(R, W2, Cf, nt, x1_ref, x2m_ref, x2p_ref, x2n_ref,
                ww_ref, wm_ref, wc_ref, s_ref, bm_ref, bt_ref,
                out_ref, win_ref, winl_ref, winr_ref):
    t = pl.program_id(1)
    TP = R * W2
    SZ = TP + 4 * W2
    zrow = jnp.zeros((Cf, W2), jnp.bfloat16)

    win_ref[:, 0:W2] = zrow
    win_ref[:, W2:2 * W2] = x2p_ref[0, :, 7, :].astype(jnp.bfloat16)
    win_ref[:, 2 * W2:2 * W2 + TP] = x2m_ref[0].reshape(Cf, TP).astype(jnp.bfloat16)
    win_ref[:, 2 * W2 + TP:3 * W2 + TP] = x2n_ref[0, :, 0, :].astype(jnp.bfloat16)
    win_ref[:, 3 * W2 + TP:SZ] = zrow

    @pl.when(t == 0)
    def _():
        win_ref[:, W2:2 * W2] = zrow

    @pl.when(t == nt - 1)
    def _():
        win_ref[:, 2 * W2 + TP:3 * W2 + TP] = zrow

    ii = jax.lax.broadcasted_iota(jnp.int32, (1, SZ - 1), 1)
    ml = (((ii + 1) % W2) != 0).astype(jnp.bfloat16)
    mr = ((ii % W2) != (W2 - 1)).astype(jnp.bfloat16)
    winl_ref[:, 1:SZ] = win_ref[:, 0:SZ - 1] * ml
    winr_ref[:, 0:SZ - 1] = win_ref[:, 1:SZ] * mr

    acc = jnp.zeros((Cf, TP), jnp.float32)
    for ky in range(3):
        base = W2 + ky * W2
        for kx, src in ((0, winl_ref), (1, win_ref), (2, winr_ref)):
            acc = acc + jnp.dot(wm_ref[ky * 3 + kx],
                                src[:, base:base + TP],
                                preferred_element_type=jnp.float32)
    med = acc + bm_ref[...]
    med = jnp.maximum(med, 0.2 * med).astype(jnp.bfloat16)
    res = jnp.dot(wc_ref[...], med,
                  preferred_element_type=jnp.float32) + bt_ref[...]

    up4 = jnp.dot(ww_ref[...], x1_ref[0].astype(jnp.bfloat16),
                  preferred_element_type=jnp.float32)
    W = W2 // 2
    rows = []
    for r in range(R):
        dy, h = r % 2, r // 2
        a = up4[(2 * dy) * Cf:(2 * dy + 1) * Cf, h * W:(h + 1) * W]
        b = up4[(2 * dy + 1) * Cf:(2 * dy + 2) * Cf, h * W:(h + 1) * W]
        cat = jnp.concatenate([a, b], axis=1).astype(jnp.bfloat16)
        rows.append(jnp.dot(cat, s_ref[...],
                            preferred_element_type=jnp.float32))
    full = res + jnp.concatenate(rows, axis=1)
    out_ref[0] = full.reshape(Cf, R, W2)


def kernel(x1, x2, w_up, b_up, w_med, b_med, w_c, b_c):
    B, Cin, H, W = x1.shape
    _, Cf, H2, W2 = x2.shape
    R = 16 if H2 % 16 == 0 and H2 > 16 else 8
    nt = H2 // R
    TP = R * W2

    wc2 = w_c[:, :, 0, 0]
    wc_med = wc2[:, :Cf].astype(jnp.bfloat16)
    wc_up = wc2[:, Cf:]
    ww = jnp.einsum('oc,kcyx->yxok', wc_up, w_up).reshape(4 * Cf, Cin)
    ww = ww.astype(jnp.bfloat16)
    wm = jnp.transpose(w_med, (2, 3, 0, 1)).reshape(9, Cf, Cf)
    wm = wm.astype(jnp.bfloat16)
    bt = (b_c + wc_up @ b_up).reshape(Cf, 1)
    bm = b_med.reshape(Cf, 1)
    rr = jnp.arange(W2)[:, None]
    cc = jnp.arange(W2)[None, :]
    s_mat = (((cc % 2 == 0) & (rr == cc // 2))
             | ((cc % 2 == 1) & (rr == W2 // 2 + cc // 2))).astype(jnp.bfloat16)

    x1f = x1.reshape(B, Cin, H * W)

    out = pl.pallas_call(
        functools.partial(_fused_body, R, W2, Cf, nt),
        out_shape=jax.ShapeDtypeStruct((B, Cf, H2, W2), jnp.float32),
        grid=(B, nt),
        in_specs=[
            pl.BlockSpec((1, Cin, (R // 2) * W), lambda b, t: (b, 0, t)),
            pl.BlockSpec((1, Cf, R, W2), lambda b, t: (b, 0, t, 0)),
            pl.BlockSpec((1, Cf, 8, W2),
                         lambda b, t: (b, 0, jnp.maximum(R * t // 8 - 1, 0), 0)),
            pl.BlockSpec((1, Cf, 8, W2),
                         lambda b, t: (b, 0,
                                       jnp.minimum(R * (t + 1) // 8,
                                                   H2 // 8 - 1), 0)),
            pl.BlockSpec((4 * Cf, Cin), lambda b, t: (0, 0)),
            pl.BlockSpec((9, Cf, Cf), lambda b, t: (0, 0, 0)),
            pl.BlockSpec((Cf, Cf), lambda b, t: (0, 0)),
            pl.BlockSpec((W2, W2), lambda b, t: (0, 0)),
            pl.BlockSpec((Cf, 1), lambda b, t: (0, 0)),
            pl.BlockSpec((Cf, 1), lambda b, t: (0, 0)),
        ],
        out_specs=pl.BlockSpec((1, Cf, R, W2), lambda b, t: (b, 0, t, 0)),
        scratch_shapes=[pltpu.VMEM((Cf, TP + 4 * W2), jnp.bfloat16),
                        pltpu.VMEM((Cf, TP + 4 * W2), jnp.bfloat16),
                        pltpu.VMEM((Cf, TP + 4 * W2), jnp.bfloat16)],
        compiler_params=pltpu.CompilerParams(
            dimension_semantics=("parallel", "parallel"),
            vmem_limit_bytes=64 * 1024 * 1024),
    )(x1f, x2, x2, x2, ww, wm, wc_med, s_mat,
      b_med.reshape(Cf, 1), bt)
    return out


# speedup vs baseline: 1.8077x; 1.0882x over previous
import functools

import jax
import jax.numpy as jnp
from jax.experimental import pallas as pl
from jax.experimental.pallas import tpu as pltpu


def _fused_body(R, W2, Cf, nt, x1_ref, x2m_ref, x2p_ref, x2n_ref,
                ww_ref, wm_ref, wc_ref, s_ref, bm_ref, bt_ref,
                out_ref, win_ref, winl_ref, winr_ref):
    t = pl.program_id(1)
    TP = R * W2
    SZ = TP + 4 * W2
    zrow = jnp.zeros((Cf, W2), jnp.bfloat16)

    win_ref[:, 0:W2] = zrow
    win_ref[:, W2:2 * W2] = x2p_ref[0, :, 7, :].astype(jnp.bfloat16)
    win_ref[:, 2 * W2:2 * W2 + TP] = x2m_ref[0].reshape(Cf, TP).astype(jnp.bfloat16)
    win_ref[:, 2 * W2 + TP:3 * W2 + TP] = x2n_ref[0, :, 0, :].astype(jnp.bfloat16)
    win_ref[:, 3 * W2 + TP:SZ] = zrow

    @pl.when(t == 0)
    def _():
        win_ref[:, W2:2 * W2] = zrow

    @pl.when(t == nt - 1)
    def _():
        win_ref[:, 2 * W2 + TP:3 * W2 + TP] = zrow

    ii = jax.lax.broadcasted_iota(jnp.int32, (1, SZ - 1), 1)
    ml = (((ii + 1) % W2) != 0).astype(jnp.bfloat16)
    mr = ((ii % W2) != (W2 - 1)).astype(jnp.bfloat16)
    winl_ref[:, 1:SZ] = win_ref[:, 0:SZ - 1] * ml
    winr_ref[:, 0:SZ - 1] = win_ref[:, 1:SZ] * mr

    acc = jnp.zeros((Cf, TP), jnp.float32)
    for ky in range(3):
        base = W2 + ky * W2
        for kx, src in ((0, winl_ref), (1, win_ref), (2, winr_ref)):
            acc = acc + jnp.dot(wm_ref[ky * 3 + kx],
                                src[:, base:base + TP],
                                preferred_element_type=jnp.float32)
    med = acc + bm_ref[...]
    med = jnp.maximum(med, 0.2 * med).astype(jnp.bfloat16)
    res = jnp.dot(wc_ref[...], med,
                  preferred_element_type=jnp.float32) + bt_ref[...]

    x1v = x1_ref[0].reshape(-1, x1_ref.shape[3]).astype(jnp.bfloat16)
    up4 = jax.lax.dot_general(ww_ref[...], x1v, (((1,), (1,)), ((), ())),
                              preferred_element_type=jnp.float32)
    W = W2 // 2
    rows = []
    for r in range(R):
        dy, h = r % 2, r // 2
        a = up4[(2 * dy) * Cf:(2 * dy + 1) * Cf, h * W:(h + 1) * W]
        b = up4[(2 * dy + 1) * Cf:(2 * dy + 2) * Cf, h * W:(h + 1) * W]
        cat = jnp.concatenate([a, b], axis=1).astype(jnp.bfloat16)
        rows.append(jnp.dot(cat, s_ref[...],
                            preferred_element_type=jnp.float32))
    full = res + jnp.concatenate(rows, axis=1)
    out_ref[0] = full.reshape(Cf, R, W2)


def kernel(x1, x2, w_up, b_up, w_med, b_med, w_c, b_c):
    B, Cin, H, W = x1.shape
    _, Cf, H2, W2 = x2.shape
    R = 16 if H2 % 16 == 0 and H2 > 16 else 8
    nt = H2 // R
    TP = R * W2

    wc2 = w_c[:, :, 0, 0]
    wc_med = wc2[:, :Cf].astype(jnp.bfloat16)
    wc_up = wc2[:, Cf:]
    ww = jnp.einsum('oc,kcyx->yxok', wc_up, w_up).reshape(4 * Cf, Cin)
    ww = ww.astype(jnp.bfloat16)
    wm = jnp.transpose(w_med, (2, 3, 0, 1)).reshape(9, Cf, Cf)
    wm = wm.astype(jnp.bfloat16)
    bt = (b_c + wc_up @ b_up).reshape(Cf, 1)
    bm = b_med.reshape(Cf, 1)
    rr = jnp.arange(W2)[:, None]
    cc = jnp.arange(W2)[None, :]
    s_mat = (((cc % 2 == 0) & (rr == cc // 2))
             | ((cc % 2 == 1) & (rr == W2 // 2 + cc // 2))).astype(jnp.bfloat16)

    x1t = jnp.transpose(x1, (0, 2, 3, 1))

    out = pl.pallas_call(
        functools.partial(_fused_body, R, W2, Cf, nt),
        out_shape=jax.ShapeDtypeStruct((B, Cf, H2, W2), jnp.float32),
        grid=(B, nt),
        in_specs=[
            pl.BlockSpec((1, R // 2, W, Cin), lambda b, t: (b, t, 0, 0)),
            pl.BlockSpec((1, Cf, R, W2), lambda b, t: (b, 0, t, 0)),
            pl.BlockSpec((1, Cf, 8, W2),
                         lambda b, t: (b, 0, jnp.maximum(R * t // 8 - 1, 0), 0)),
            pl.BlockSpec((1, Cf, 8, W2),
                         lambda b, t: (b, 0,
                                       jnp.minimum(R * (t + 1) // 8,
                                                   H2 // 8 - 1), 0)),
            pl.BlockSpec((4 * Cf, Cin), lambda b, t: (0, 0)),
            pl.BlockSpec((9, Cf, Cf), lambda b, t: (0, 0, 0)),
            pl.BlockSpec((Cf, Cf), lambda b, t: (0, 0)),
            pl.BlockSpec((W2, W2), lambda b, t: (0, 0)),
            pl.BlockSpec((Cf, 1), lambda b, t: (0, 0)),
            pl.BlockSpec((Cf, 1), lambda b, t: (0, 0)),
        ],
        out_specs=pl.BlockSpec((1, Cf, R, W2), lambda b, t: (b, 0, t, 0)),
        scratch_shapes=[pltpu.VMEM((Cf, TP + 4 * W2), jnp.bfloat16),
                        pltpu.VMEM((Cf, TP + 4 * W2), jnp.bfloat16),
                        pltpu.VMEM((Cf, TP + 4 * W2), jnp.bfloat16)],
        compiler_params=pltpu.CompilerParams(
            dimension_semantics=("parallel", "parallel"),
            vmem_limit_bytes=64 * 1024 * 1024),
    )(x1t, x2, x2, x2, ww, wm, wc_med, s_mat,
      b_med.reshape(Cf, 1), bt)
    return out


# speedup vs baseline: 2.0804x; 1.1509x over previous
import functools

import jax
import jax.numpy as jnp
from jax.experimental import pallas as pl
from jax.experimental.pallas import tpu as pltpu


def _fused_body(R, W2, Cf, nt, x1_ref, x2m_ref, x2p_ref, x2n_ref,
                ww_ref, wm_ref, wc_ref, s_ref, bm_ref, bt_ref,
                out_ref, win_ref, winl_ref, winr_ref):
    t = pl.program_id(1)
    TP = R * W2
    SZ = TP + 4 * W2
    zrow = jnp.zeros((Cf, W2), jnp.bfloat16)

    win_ref[:, 0:W2] = zrow
    win_ref[:, W2:2 * W2] = x2p_ref[0, :, 7, :].astype(jnp.bfloat16)
    win_ref[:, 2 * W2:2 * W2 + TP] = x2m_ref[0].reshape(Cf, TP).astype(jnp.bfloat16)
    win_ref[:, 2 * W2 + TP:3 * W2 + TP] = x2n_ref[0, :, 0, :].astype(jnp.bfloat16)
    win_ref[:, 3 * W2 + TP:SZ] = zrow

    @pl.when(t == 0)
    def _():
        win_ref[:, W2:2 * W2] = zrow

    @pl.when(t == nt - 1)
    def _():
        win_ref[:, 2 * W2 + TP:3 * W2 + TP] = zrow

    ii = jax.lax.broadcasted_iota(jnp.int32, (1, SZ - 1), 1)
    ml = (((ii + 1) % W2) != 0).astype(jnp.bfloat16)
    mr = ((ii % W2) != (W2 - 1)).astype(jnp.bfloat16)
    winl_ref[:, 1:SZ] = win_ref[:, 0:SZ - 1] * ml
    winr_ref[:, 0:SZ - 1] = win_ref[:, 1:SZ] * mr

    acc = jnp.zeros((Cf, TP), jnp.float32)
    for ky in range(3):
        base = W2 + ky * W2
        for kx, src in ((0, winl_ref), (1, win_ref), (2, winr_ref)):
            acc = acc + jnp.dot(wm_ref[ky * 3 + kx],
                                src[:, base:base + TP],
                                preferred_element_type=jnp.float32)
    med = acc + bm_ref[...]
    med = jnp.maximum(med, 0.2 * med).astype(jnp.bfloat16)
    res = jnp.dot(wc_ref[...], med,
                  preferred_element_type=jnp.float32) + bt_ref[...]

    x1v = x1_ref[0].reshape(-1, x1_ref.shape[3]).astype(jnp.bfloat16)
    up4 = jax.lax.dot_general(ww_ref[...], x1v, (((1,), (1,)), ((), ())),
                              preferred_element_type=jnp.float32)
    W = W2 // 2
    rows = []
    for r in range(R):
        dy, h = r % 2, r // 2
        a = up4[(2 * dy) * Cf:(2 * dy + 1) * Cf, h * W:(h + 1) * W]
        b = up4[(2 * dy + 1) * Cf:(2 * dy + 2) * Cf, h * W:(h + 1) * W]
        cat = jnp.concatenate([a, b], axis=1).astype(jnp.bfloat16)
        rows.append(jnp.dot(cat, s_ref[...],
                            preferred_element_type=jnp.float32))
    full = res + jnp.concatenate(rows, axis=1)
    out_ref[0] = full.reshape(Cf, R, W2)


def kernel(x1, x2, w_up, b_up, w_med, b_med, w_c, b_c):
    B, Cin, H, W = x1.shape
    _, Cf, H2, W2 = x2.shape
    R = 32 if H2 % 32 == 0 and H2 > 32 else 8
    nt = H2 // R
    TP = R * W2

    wc2 = w_c[:, :, 0, 0]
    wc_med = wc2[:, :Cf].astype(jnp.bfloat16)
    wc_up = wc2[:, Cf:]
    ww = jnp.einsum('oc,kcyx->yxok', wc_up, w_up).reshape(4 * Cf, Cin)
    ww = ww.astype(jnp.bfloat16)
    wm = jnp.transpose(w_med, (2, 3, 0, 1)).reshape(9, Cf, Cf)
    wm = wm.astype(jnp.bfloat16)
    bt = (b_c + wc_up @ b_up).reshape(Cf, 1)
    bm = b_med.reshape(Cf, 1)
    rr = jnp.arange(W2)[:, None]
    cc = jnp.arange(W2)[None, :]
    s_mat = (((cc % 2 == 0) & (rr == cc // 2))
             | ((cc % 2 == 1) & (rr == W2 // 2 + cc // 2))).astype(jnp.bfloat16)

    x1t = jnp.transpose(x1, (0, 2, 3, 1))

    out = pl.pallas_call(
        functools.partial(_fused_body, R, W2, Cf, nt),
        out_shape=jax.ShapeDtypeStruct((B, Cf, H2, W2), jnp.float32),
        grid=(B, nt),
        in_specs=[
            pl.BlockSpec((1, R // 2, W, Cin), lambda b, t: (b, t, 0, 0)),
            pl.BlockSpec((1, Cf, R, W2), lambda b, t: (b, 0, t, 0)),
            pl.BlockSpec((1, Cf, 8, W2),
                         lambda b, t: (b, 0, jnp.maximum(R * t // 8 - 1, 0), 0)),
            pl.BlockSpec((1, Cf, 8, W2),
                         lambda b, t: (b, 0,
                                       jnp.minimum(R * (t + 1) // 8,
                                                   H2 // 8 - 1), 0)),
            pl.BlockSpec((4 * Cf, Cin), lambda b, t: (0, 0)),
            pl.BlockSpec((9, Cf, Cf), lambda b, t: (0, 0, 0)),
            pl.BlockSpec((Cf, Cf), lambda b, t: (0, 0)),
            pl.BlockSpec((W2, W2), lambda b, t: (0, 0)),
            pl.BlockSpec((Cf, 1), lambda b, t: (0, 0)),
            pl.BlockSpec((Cf, 1), lambda b, t: (0, 0)),
        ],
        out_specs=pl.BlockSpec((1, Cf, R, W2), lambda b, t: (b, 0, t, 0)),
        scratch_shapes=[pltpu.VMEM((Cf, TP + 4 * W2), jnp.bfloat16),
                        pltpu.VMEM((Cf, TP + 4 * W2), jnp.bfloat16),
                        pltpu.VMEM((Cf, TP + 4 * W2), jnp.bfloat16)],
        compiler_params=pltpu.CompilerParams(
            dimension_semantics=("parallel", "parallel"),
            vmem_limit_bytes=64 * 1024 * 1024),
    )(x1t, x2, x2, x2, ww, wm, wc_med, s_mat,
      b_med.reshape(Cf, 1), bt)
    return out
